# Initial kernel scaffold; baseline (speedup 1.0000x reference)
import jax
import jax.numpy as jnp
from jax.experimental import pallas as pl


def kernel(x, edge_index, W1, b1, Wq0, bq0, Wk0, bk0, Wv0, bv0, Wq1, bq1, Wk1, bk1, Wv1, bv1, W2, b2):
    raise NotImplementedError("write your pallas kernel here")


# speedup vs baseline: 1.1358x; 1.1358x over previous
import functools
import math

import jax
import jax.numpy as jnp
from jax.experimental import pallas as pl
from jax.experimental.pallas import tpu as pltpu

N = 10000
E = 160000
C = 128
HEADS = 8
D = C // HEADS
NP = 10240


def _proj_kernel(x_ref, w_ref, b_ref, o_ref, *, relu_in, nout):
    h = x_ref[...]
    if relu_in:
        h = jnp.maximum(h, 0.0)
    acc = jnp.dot(h, w_ref[...], preferred_element_type=jnp.float32)
    o_ref[...] = acc + b_ref[...]


def _fused_projs(z, Ws, bs, relu_in, blk=1024):
    nw = len(Ws)
    Wcat = jnp.concatenate(Ws, axis=1)
    bcat = jnp.concatenate(bs, axis=0)
    cout = Wcat.shape[1]
    out = pl.pallas_call(
        functools.partial(_proj_kernel, relu_in=relu_in, nout=cout),
        grid=(NP // blk,),
        in_specs=[
            pl.BlockSpec((blk, z.shape[1]), lambda i: (i, 0)),
            pl.BlockSpec((z.shape[1], cout), lambda i: (0, 0)),
            pl.BlockSpec((cout,), lambda i: (0,)),
        ],
        out_specs=pl.BlockSpec((blk, cout), lambda i: (i, 0)),
        out_shape=jax.ShapeDtypeStruct((NP, cout), jnp.float32),
    )(z, Wcat, bcat)
    return [out[:, i * C:(i + 1) * C] for i in range(nw)] if nw > 1 else [out]


def _restricted_softmax(s):
    smax = jnp.maximum(jnp.max(s, axis=-1, keepdims=True), 0.0)
    num = jnp.exp(s - smax)
    return num / (jnp.sum(num, axis=-1, keepdims=True) + jnp.exp(-smax))


def _edge_conv(Q, Ks, Vs, src, dst, norm):
    L = len(Ks)
    q = Q[dst].reshape(-1, HEADS, D)
    k = jnp.stack([K[src] for K in Ks], 1).reshape(-1, L, HEADS, D)
    v = jnp.stack([V[src] for V in Vs], 1).reshape(-1, L, HEADS, D)
    score = jnp.einsum('ehd,elhd->ehl', q, k) / math.sqrt(D)
    attn = _restricted_softmax(score)
    out = jnp.einsum('ehl,elhd->ehd', attn, v).reshape(-1, C)
    msg = norm[:, None] * out
    return jax.ops.segment_sum(msg, dst, num_segments=N)


def _out_kernel(z_ref, w_ref, b_ref, o_ref):
    h = jnp.maximum(z_ref[...], 0.0)
    o = jnp.dot(h, w_ref[...], preferred_element_type=jnp.float32) + b_ref[...]
    m = jnp.max(o, axis=-1, keepdims=True)
    ex = jnp.exp(o - m)
    o_ref[...] = o - m - jnp.log(jnp.sum(ex, axis=-1, keepdims=True))


def kernel(x, edge_index, W1, b1, Wq0, bq0, Wk0, bk0, Wv0, bv0, Wq1, bq1, Wk1, bk1, Wv1, bv1, W2, b2):
    src, dst = edge_index[0], edge_index[1]
    loop = jnp.arange(N, dtype=jnp.int32)
    src2 = jnp.concatenate([src, loop])
    dst2 = jnp.concatenate([dst, loop])
    w2 = jnp.concatenate([(src != dst).astype(jnp.float32), jnp.ones((N,), jnp.float32)])
    deg = jax.ops.segment_sum(w2, dst2, num_segments=N)
    dinv = jnp.where(deg > 0, 1.0 / jnp.sqrt(deg), 0.0)
    norm = dinv[src2] * w2 * dinv[dst2]

    xp = jnp.pad(x, ((0, NP - N), (0, 0)))
    h_, q0_, k0_, v0_, k1a_, v1a_ = _fused_projs(
        xp @ W1 + b1,
        [jnp.eye(C, dtype=jnp.float32), Wq0, Wk0, Wv0, Wk1, Wv1],
        [jnp.zeros((C,), jnp.float32), bq0, bk0, bv0, bk1, bv1],
        relu_in=True)
    agg0 = _edge_conv(q0_[:N], [k0_[:N]], [v0_[:N]], src2, dst2, norm)

    agg0p = jnp.pad(agg0, ((0, NP - N), (0, 0)))
    q1_, k1b_, v1b_ = _fused_projs(agg0p, [Wq1, Wk1, Wv1], [bq1, bk1, bv1], relu_in=True)
    agg1 = _edge_conv(q1_[:N], [k1a_[:N], k1b_[:N]], [v1a_[:N], v1b_[:N]], src2, dst2, norm)

    agg1p = jnp.pad(agg1, ((0, NP - N), (0, 0)))
    out = pl.pallas_call(
        _out_kernel,
        grid=(NP // 1024,),
        in_specs=[
            pl.BlockSpec((1024, C), lambda i: (i, 0)),
            pl.BlockSpec((C, 64), lambda i: (0, 0)),
            pl.BlockSpec((64,), lambda i: (0,)),
        ],
        out_specs=pl.BlockSpec((1024, 64), lambda i: (i, 0)),
        out_shape=jax.ShapeDtypeStruct((NP, 64), jnp.float32),
    )(agg1p, W2, b2)
    return out[:N]


# speedup vs baseline: 2.4408x; 2.1491x over previous
import functools

import jax
import jax.numpy as jnp
import numpy as np
from jax import lax
from jax.experimental import pallas as pl
from jax.experimental.pallas import tpu as pltpu
from jax.experimental.pallas import tpu_sc as plsc

N = 10000
E = 160000
C = 128
HEADS = 8
D = C // HEADS
NP = 10240
NTILES = 32
B = 64
E2 = E + N
NBLK = -(-E2 // (NTILES * B))
E2P = NTILES * B * NBLK
TPT = E2P // NTILES
ZR = NP // 16

_PERM = (np.arange(C) % HEADS) * D + np.arange(C) // HEADS


_DNUMS = lax.GatherDimensionNumbers(
    offset_dims=(), collapsed_slice_dims=(0,), start_index_map=(0,))


def _lane_gather(v, idx):
    return lax.gather(v, idx[:, None], _DNUMS, slice_sizes=(1,),
                      mode=lax.GatherScatterMode.PROMISE_IN_BOUNDS)


def _rot8(v):
    return _lane_gather(v, lax.iota(jnp.int32, 16) ^ 8)


def _splat_lane(v, i):
    return _lane_gather(v, jnp.full((16,), i, jnp.int32))


def _sc_conv(L):
    mesh = plsc.VectorSubcoreMesh(core_axis_name="c", subcore_axis_name="s",
                                  num_cores=2)
    scratch = (
        [pltpu.VMEM_SHARED((NP, C), jnp.float32)] +
        [pltpu.VMEM((B,), jnp.int32)] * 2 +
        [pltpu.VMEM((B,), jnp.float32)] +
        [pltpu.VMEM((B, C), jnp.float32)] * (2 * L + 1) +
        [pltpu.SemaphoreType.DMA]
    )

    @functools.partial(
        pl.kernel, mesh=mesh,
        out_type=jax.ShapeDtypeStruct((2, NP, C), jnp.float32),
        scratch_types=scratch)
    def body(*refs):
        (qt_hbm, *kv_hbm), rest = refs[:1 + 2 * L], refs[1 + 2 * L:]
        kt_hbm, vt_hbm = kv_hbm[:L], kv_hbm[L:]
        src_hbm, dst_hbm, norm_hbm, zero_hbm, out_hbm = rest[:5]
        agg_sh, src_v, dst_v, norm_v = rest[5:9]
        q_rows = rest[9]
        k_rows = rest[10:10 + L]
        v_rows = rest[10 + L:10 + 2 * L]
        msg_v = q_rows
        sem = rest[10 + 2 * L]

        cid = lax.axis_index("c")
        sid = lax.axis_index("s")
        wid = cid * 16 + sid

        pltpu.sync_copy(zero_hbm, agg_sh.at[pl.ds(sid * ZR, ZR)])
        plsc.subcore_barrier()

        def block(b, _):
            base = wid * TPT + b * B
            pltpu.sync_copy(src_hbm.at[pl.ds(base, B)], src_v)
            pltpu.sync_copy(dst_hbm.at[pl.ds(base, B)], dst_v)
            pltpu.sync_copy(norm_hbm.at[pl.ds(base, B)], norm_v)
            cps = [pltpu.async_copy(qt_hbm.at[dst_v], q_rows, sem)]
            for l in range(L):
                cps.append(pltpu.async_copy(kt_hbm[l].at[src_v], k_rows[l], sem))
                cps.append(pltpu.async_copy(vt_hbm[l].at[src_v], v_rows[l], sem))
            for cp in cps:
                cp.wait()

            def group(g, _):
                nb = norm_v[pl.ds(g * 16, 16)]
                for ei in range(16):
                    e = g * 16 + ei
                    qv = [q_rows[e, pl.ds(16 * jp, 16)] for jp in range(8)]
                    svecs = []
                    for l in range(L):
                        acc = qv[0] * k_rows[l][e, pl.ds(0, 16)]
                        for jp in range(1, 8):
                            acc = acc + qv[jp] * k_rows[l][e, pl.ds(16 * jp, 16)]
                        svecs.append((acc + _rot8(acc)) * 0.25)
                    m = svecs[0]
                    for l in range(1, L):
                        m = jnp.maximum(m, svecs[l])
                    m = jnp.maximum(m, 0.0)
                    es = [jnp.exp(s - m) for s in svecs]
                    den = jnp.exp(-m)
                    for e_l in es:
                        den = den + e_l
                    scale = _splat_lane(nb, ei) / den
                    coef = [e_l * scale for e_l in es]
                    for jp in range(8):
                        o = coef[0] * v_rows[0][e, pl.ds(16 * jp, 16)]
                        for l in range(1, L):
                            o = o + coef[l] * v_rows[l][e, pl.ds(16 * jp, 16)]
                        msg_v[e, pl.ds(16 * jp, 16)] = o
                return 0

            lax.fori_loop(0, B // 16, group, 0)
            pltpu.sync_copy(msg_v, agg_sh.at[dst_v], add=True)
            return 0

        lax.fori_loop(0, NBLK, block, 0)
        plsc.subcore_barrier()
        pltpu.sync_copy(agg_sh.at[pl.ds(sid * ZR, ZR)],
                        out_hbm.at[cid, pl.ds(sid * ZR, ZR)])

    return body


_sc_conv1 = _sc_conv(1)
_sc_conv2 = _sc_conv(2)


def _proj_kernel(nin, has0, z_ref, *rest):
    if has0:
        w0_ref, b0_ref, w_ref, b_ref, o_ref = rest
    else:
        w_ref, b_ref, o_ref = rest
    h = z_ref[0] + z_ref[1] if nin == 2 else z_ref[0]
    if has0:
        h = jnp.dot(h, w0_ref[...],
                    preferred_element_type=jnp.float32) + b0_ref[...]
    h = jnp.maximum(h, 0.0)
    o_ref[...] = jnp.dot(h, w_ref[...],
                         preferred_element_type=jnp.float32) + b_ref[...]


def _fused_projs(z, Ws, bs, W0=None, b0=None, blk=2048):
    nin = z.shape[0]
    Wcat = jnp.concatenate(Ws, axis=1)
    bcat = jnp.concatenate(bs, axis=0)
    cout = Wcat.shape[1]
    has0 = W0 is not None
    in_specs = [pl.BlockSpec((nin, blk, C), lambda i: (0, i, 0))]
    args = [z]
    if has0:
        in_specs += [pl.BlockSpec((C, C), lambda i: (0, 0)),
                     pl.BlockSpec((C,), lambda i: (0,))]
        args += [W0, b0]
    in_specs += [pl.BlockSpec((C, cout), lambda i: (0, 0)),
                 pl.BlockSpec((cout,), lambda i: (0,))]
    args += [Wcat, bcat]
    out = pl.pallas_call(
        functools.partial(_proj_kernel, nin, has0),
        grid=(NP // blk,),
        in_specs=in_specs,
        out_specs=pl.BlockSpec((blk, cout), lambda i: (i, 0)),
        out_shape=jax.ShapeDtypeStruct((NP, cout), jnp.float32),
    )(*args)
    return [out[:, i * C:(i + 1) * C] for i in range(len(Ws))]


def _out_kernel(z_ref, w_ref, b_ref, o_ref):
    h = jnp.maximum(z_ref[0] + z_ref[1], 0.0)
    o = jnp.dot(h, w_ref[...], preferred_element_type=jnp.float32) + b_ref[...]
    m = jnp.max(o, axis=-1, keepdims=True)
    ex = jnp.exp(o - m)
    o_ref[...] = o - m - jnp.log(jnp.sum(ex, axis=-1, keepdims=True))


def kernel(x, edge_index, W1, b1, Wq0, bq0, Wk0, bk0, Wv0, bv0, Wq1, bq1, Wk1, bk1, Wv1, bv1, W2, b2):
    src, dst = edge_index[0], edge_index[1]
    loop = jnp.arange(N, dtype=jnp.int32)
    pad = jnp.zeros((E2P - E2,), jnp.int32)
    src2 = jnp.concatenate([src, loop, pad])
    dst2 = jnp.concatenate([dst, loop, pad])
    w2 = jnp.concatenate([(src != dst).astype(jnp.float32),
                          jnp.ones((N,), jnp.float32)])
    deg = jax.ops.segment_sum(w2, dst2[:E2], num_segments=N)
    dinv = jnp.where(deg > 0, 1.0 / jnp.sqrt(deg), 0.0)
    norm = jnp.concatenate(
        [dinv[src2[:E2]] * w2 * dinv[dst2[:E2]],
         jnp.zeros((E2P - E2,), jnp.float32)])

    p = _PERM
    zero_z = jnp.zeros((ZR, C), jnp.float32)
    xp = jnp.pad(x, ((0, NP - N), (0, 0)))
    qt0, kt0, vt0, kt1a, vt1a = _fused_projs(
        xp[None],
        [Wq0[:, p], Wk0[:, p], Wv0[:, p], Wk1[:, p], Wv1[:, p]],
        [bq0[p], bk0[p], bv0[p], bk1[p], bv1[p]],
        W0=W1, b0=b1)
    agg0 = _sc_conv1(qt0, kt0, vt0, src2, dst2, norm, zero_z)

    qt1, kt1b, vt1b = _fused_projs(
        agg0,
        [Wq1[p][:, p], Wk1[p][:, p], Wv1[p][:, p]],
        [bq1[p], bk1[p], bv1[p]])
    agg1 = _sc_conv2(qt1, kt1a, kt1b, vt1a, vt1b, src2, dst2, norm, zero_z)

    out = pl.pallas_call(
        _out_kernel,
        grid=(NP // 2048,),
        in_specs=[
            pl.BlockSpec((2, 2048, C), lambda i: (0, i, 0)),
            pl.BlockSpec((C, 64), lambda i: (0, 0)),
            pl.BlockSpec((64,), lambda i: (0,)),
        ],
        out_specs=pl.BlockSpec((2048, 64), lambda i: (i, 0)),
        out_shape=jax.ShapeDtypeStruct((NP, 64), jnp.float32),
    )(agg1, W2[p], b2)
    return out[:N]


# speedup vs baseline: 6.3154x; 2.5874x over previous
import functools

import jax
import jax.numpy as jnp
import numpy as np
from jax import lax
from jax.experimental import pallas as pl
from jax.experimental.pallas import tpu as pltpu
from jax.experimental.pallas import tpu_sc as plsc

N = 10000
E = 160000
C = 128
HEADS = 8
D = C // HEADS
NP = 10240
NTILES = 32
B = 64
E2 = E + N
NBLK = -(-E2 // (NTILES * B))
E2P = NTILES * B * NBLK
TPT = E2P // NTILES
ZR = NP // 16

_PERM = (np.arange(C) % HEADS) * D + np.arange(C) // HEADS

_DNUMS = lax.GatherDimensionNumbers(
    offset_dims=(), collapsed_slice_dims=(0,), start_index_map=(0,))


def _lane_gather(v, idx):
    return lax.gather(v, idx[:, None], _DNUMS, slice_sizes=(1,),
                      mode=lax.GatherScatterMode.PROMISE_IN_BOUNDS)


def _rot8(v):
    return _lane_gather(v, lax.iota(jnp.int32, 16) ^ 8)


def _splat_lane(v, i):
    return _lane_gather(v, jnp.full((16,), i, jnp.int32))


def _sc_conv(L):
    calc_norm = L == 1
    mesh = plsc.VectorSubcoreMesh(core_axis_name="c", subcore_axis_name="s",
                                  num_cores=2)
    scratch = (
        [pltpu.VMEM_SHARED((NP, C), jnp.float32)] +
        [pltpu.VMEM((B,), jnp.int32)] * 2 +
        [pltpu.VMEM((B,), jnp.float32)] +
        ([pltpu.VMEM((B,), jnp.float32)] * 2 if calc_norm else []) +
        [pltpu.VMEM((B, C), jnp.float32)] * (2 * L + 1) +
        [pltpu.SemaphoreType.DMA]
    )
    out_type = [jax.ShapeDtypeStruct((2, NP, C), jnp.float32)]
    if calc_norm:
        out_type.append(jax.ShapeDtypeStruct((E2P,), jnp.float32))

    @functools.partial(
        pl.kernel, mesh=mesh, out_type=tuple(out_type),
        scratch_types=scratch)
    def body(*refs):
        it = iter(refs)
        qt_hbm = next(it)
        kt_hbm = [next(it) for _ in range(L)]
        vt_hbm = [next(it) for _ in range(L)]
        src_hbm, dst_hbm = next(it), next(it)
        dinv_hbm = next(it) if calc_norm else None
        norm_hbm = None if calc_norm else next(it)
        zero_hbm = next(it)
        out_hbm = next(it)
        normout_hbm = next(it) if calc_norm else None
        agg_sh, src_v, dst_v, norm_v = next(it), next(it), next(it), next(it)
        if calc_norm:
            gs_v, gd_v = next(it), next(it)
        q_rows = next(it)
        k_rows = [next(it) for _ in range(L)]
        v_rows = [next(it) for _ in range(L)]
        msg_v = q_rows
        sem = next(it)

        cid = lax.axis_index("c")
        sid = lax.axis_index("s")
        wid = cid * 16 + sid

        pltpu.sync_copy(zero_hbm, agg_sh.at[pl.ds(sid * ZR, ZR)])
        plsc.subcore_barrier()

        def block(b, _):
            base = wid * TPT + b * B
            pltpu.sync_copy(src_hbm.at[pl.ds(base, B)], src_v)
            pltpu.sync_copy(dst_hbm.at[pl.ds(base, B)], dst_v)
            if not calc_norm:
                pltpu.sync_copy(norm_hbm.at[pl.ds(base, B)], norm_v)
            cps = [pltpu.async_copy(qt_hbm.at[dst_v], q_rows, sem)]
            for l in range(L):
                cps.append(pltpu.async_copy(kt_hbm[l].at[src_v], k_rows[l], sem))
                cps.append(pltpu.async_copy(vt_hbm[l].at[src_v], v_rows[l], sem))
            if calc_norm:
                pltpu.sync_copy(dinv_hbm.at[src_v], gs_v)
                pltpu.sync_copy(dinv_hbm.at[dst_v], gd_v)

                def norm_grp(g, _):
                    s16 = src_v[pl.ds(g * 16, 16)]
                    d16 = dst_v[pl.ds(g * 16, 16)]
                    nrm = gs_v[pl.ds(g * 16, 16)] * gd_v[pl.ds(g * 16, 16)]
                    eid = jnp.full((16,), base + g * 16, jnp.int32) + \
                        lax.iota(jnp.int32, 16)
                    wf = jnp.where(s16 != d16, nrm, 0.0)
                    wf = jnp.where(eid >= E, nrm, wf)
                    wf = jnp.where(eid < E2, wf, 0.0)
                    norm_v[pl.ds(g * 16, 16)] = wf
                    return 0
                lax.fori_loop(0, B // 16, norm_grp, 0)
                pltpu.sync_copy(norm_v, normout_hbm.at[pl.ds(base, B)])
            for cp in cps:
                cp.wait()

            def group(g, _):
                nb = norm_v[pl.ds(g * 16, 16)]
                for ei in range(16):
                    e = g * 16 + ei
                    qv = [q_rows[e, pl.ds(16 * jp, 16)] for jp in range(8)]
                    svecs = []
                    for l in range(L):
                        acc = qv[0] * k_rows[l][e, pl.ds(0, 16)]
                        for jp in range(1, 8):
                            acc = acc + qv[jp] * k_rows[l][e, pl.ds(16 * jp, 16)]
                        svecs.append((acc + _rot8(acc)) * 0.25)
                    m = svecs[0]
                    for l in range(1, L):
                        m = jnp.maximum(m, svecs[l])
                    m = jnp.maximum(m, 0.0)
                    es = [jnp.exp(s - m) for s in svecs]
                    den = jnp.exp(-m)
                    for e_l in es:
                        den = den + e_l
                    scale = _splat_lane(nb, ei) / den
                    coef = [e_l * scale for e_l in es]
                    for jp in range(8):
                        o = coef[0] * v_rows[0][e, pl.ds(16 * jp, 16)]
                        for l in range(1, L):
                            o = o + coef[l] * v_rows[l][e, pl.ds(16 * jp, 16)]
                        msg_v[e, pl.ds(16 * jp, 16)] = o
                return 0

            lax.fori_loop(0, B // 16, group, 0)
            pltpu.sync_copy(msg_v, agg_sh.at[dst_v], add=True)
            return 0

        lax.fori_loop(0, NBLK, block, 0)
        plsc.subcore_barrier()
        pltpu.sync_copy(agg_sh.at[pl.ds(sid * ZR, ZR)],
                        out_hbm.at[cid, pl.ds(sid * ZR, ZR)])

    return body


_sc_conv1 = _sc_conv(1)
_sc_conv2 = _sc_conv(2)

BLK = 2000


def _proj_kernel(nin, has0, nout, *refs):
    z_ref = refs[0]
    rest = refs[1:]
    if has0:
        w0_ref, b0_ref = rest[0], rest[1]
        rest = rest[2:]
    w_ref, b_ref = rest[0], rest[1]
    o_refs = rest[2:]
    h = z_ref[0] + z_ref[1] if nin == 2 else z_ref[0]
    if has0:
        h = jnp.dot(h, w0_ref[...],
                    preferred_element_type=jnp.float32) + b0_ref[...]
    h = jnp.maximum(h, 0.0)
    acc = jnp.dot(h, w_ref[...], preferred_element_type=jnp.float32) + b_ref[...]
    for i in range(nout):
        o_refs[i][...] = acc[:, i * C:(i + 1) * C]


def _fused_projs(z, Ws, bs, W0=None, b0=None):
    nin = z.shape[0]
    nout = len(Ws)
    Wcat = jnp.concatenate(Ws, axis=1)
    bcat = jnp.concatenate(bs, axis=0)
    cout = Wcat.shape[1]
    has0 = W0 is not None
    in_specs = [pl.BlockSpec((nin, BLK, C), lambda i: (0, i, 0))]
    args = [z]
    if has0:
        in_specs += [pl.BlockSpec((C, C), lambda i: (0, 0)),
                     pl.BlockSpec((C,), lambda i: (0,))]
        args += [W0, b0]
    in_specs += [pl.BlockSpec((C, cout), lambda i: (0, 0)),
                 pl.BlockSpec((cout,), lambda i: (0,))]
    args += [Wcat, bcat]
    return pl.pallas_call(
        functools.partial(_proj_kernel, nin, has0, nout),
        grid=(N // BLK,),
        in_specs=in_specs,
        out_specs=[pl.BlockSpec((BLK, C), lambda i: (i, 0))] * nout,
        out_shape=[jax.ShapeDtypeStruct((N, C), jnp.float32)] * nout,
    )(*args)


def _out_kernel(z_ref, w_ref, b_ref, o_ref):
    h = jnp.maximum(z_ref[0] + z_ref[1], 0.0)
    o = jnp.dot(h, w_ref[...], preferred_element_type=jnp.float32) + b_ref[...]
    m = jnp.max(o, axis=-1, keepdims=True)
    ex = jnp.exp(o - m)
    o_ref[...] = o - m - jnp.log(jnp.sum(ex, axis=-1, keepdims=True))


def kernel(x, edge_index, W1, b1, Wq0, bq0, Wk0, bk0, Wv0, bv0, Wq1, bq1, Wk1, bk1, Wv1, bv1, W2, b2):
    src, dst = edge_index[0], edge_index[1]
    loop = jnp.arange(N, dtype=jnp.int32)
    pad = jnp.zeros((E2P - E2,), jnp.int32)
    src2 = jnp.concatenate([src, loop, pad])
    dst2 = jnp.concatenate([dst, loop, pad])
    w2 = jnp.concatenate([(src != dst).astype(jnp.float32),
                          jnp.ones((N,), jnp.float32)])
    deg = jax.ops.segment_sum(w2, dst2[:E2], num_segments=N)
    dinv = jnp.where(deg > 0, 1.0 / jnp.sqrt(deg), 0.0)

    p = _PERM
    zero_z = jnp.zeros((ZR, C), jnp.float32)
    qt0, kt0, vt0, kt1a, vt1a = _fused_projs(
        x[None],
        [Wq0[:, p], Wk0[:, p], Wv0[:, p], Wk1[:, p], Wv1[:, p]],
        [bq0[p], bk0[p], bv0[p], bk1[p], bv1[p]],
        W0=W1, b0=b1)
    agg0, norm = _sc_conv1(qt0, kt0, vt0, src2, dst2, dinv, zero_z)

    qt1, kt1b, vt1b = _fused_projs(
        agg0,
        [Wq1[p][:, p], Wk1[p][:, p], Wv1[p][:, p]],
        [bq1[p], bk1[p], bv1[p]])
    agg1, = _sc_conv2(qt1, kt1a, kt1b, vt1a, vt1b, src2, dst2, norm, zero_z)

    return pl.pallas_call(
        _out_kernel,
        grid=(N // BLK,),
        in_specs=[
            pl.BlockSpec((2, BLK, C), lambda i: (0, i, 0)),
            pl.BlockSpec((C, 64), lambda i: (0, 0)),
            pl.BlockSpec((64,), lambda i: (0,)),
        ],
        out_specs=pl.BlockSpec((BLK, 64), lambda i: (i, 0)),
        out_shape=jax.ShapeDtypeStruct((N, 64), jnp.float32),
    )(agg1, W2[p], b2)
